# baseline (speedup 1.0000x reference)
import jax
import jax.numpy as jnp
from jax import lax
from jax.experimental import pallas as pl
from jax.experimental.pallas import tpu as pltpu
from jax.experimental.pallas import tpu_sc as plsc

B = 16384
D = 64
C = 100000
NS = 16
SPT = B // NS
SB = 64
NSB = SPT // SB
HIST_N = 102400
TR = 25088
DUMMY = 25000
CHUNK = 25000
LW = 0.005


def _body(y_hbm, feat_hbm, centers_hbm, grad_hbm, loss_hbm,
          hist, table, lsum,
          y2d, hvm, scale2d, idx, feat_st, g_st, val_sb,
          zb1, zb2, ones, zidx, zf32, lread, lout, sem, zsem):
    c = lax.axis_index("c")
    s = lax.axis_index("s")
    lo_a = c * (2 * CHUNK)

    def zf1(k, carry):
        zb1[pl.ds(k * 16, 16)] = jnp.zeros((16,), jnp.int32)
        return carry
    lax.fori_loop(0, 800 // 16, zf1, 0)

    def zf2(t, carry):
        r = t // 4
        q = (t % 4) * 16
        zb2[r, pl.ds(q, 16)] = jnp.zeros((16,), jnp.float32)
        return carry
    lax.fori_loop(0, 32 * 4, zf2, 0)

    def of(k, carry):
        ones[pl.ds(k * 16, 16)] = jnp.ones((16,), jnp.int32)
        return carry
    lax.fori_loop(0, SB // 16, of, 0)

    zidx[pl.ds(0, 16)] = jnp.zeros((16,), jnp.int32)
    zf32[pl.ds(0, 16)] = jnp.zeros((16,), jnp.float32)

    @pl.when(s == 0)
    def _():
        pltpu.sync_copy(zf32, lsum)

    hcps = [pltpu.async_copy(zb1, hist.at[pl.ds(s * 6400 + k * 800, 800)],
                             zsem) for k in range(8)]
    for cp2 in hcps:
        cp2.wait()

    pltpu.sync_copy(y_hbm.at[pl.ds(s * NSB, NSB), :], y2d)

    plsc.subcore_barrier()

    with jax.named_scope("histadd"):
        for j in range(NSB):
            pltpu.sync_copy(ones, hist.at[y2d.at[j]], add=True)

    plsc.subcore_barrier()

    with jax.named_scope("histgat"):
        for j in range(NSB):
            pltpu.sync_copy(hist.at[y2d.at[j]], hvm.at[j])

    def fcomp(t, carry):
        j = t // 4
        q = (t % 4) * 16
        hv = hvm[j, pl.ds(q, 16)]
        scale2d[j, pl.ds(q, 16)] = 1.0 / (1.0 + hv.astype(jnp.float32))
        return carry
    lax.fori_loop(0, (NSB * SB) // 16, fcomp, 0)

    lacc = jnp.zeros((16,), jnp.float32)
    for phase in range(2):
        lo = lo_a + phase * CHUNK

        with jax.named_scope("ztab"):
            zcps = [pltpu.async_copy(zb2,
                                     table.at[pl.ds(s * 1568 + k * 32, 32), :],
                                     zsem) for k in range(49)]
            for cp2 in zcps:
                cp2.wait()

        def icomp(t, carry):
            j = t // 4
            q = (t % 4) * 16
            yv = y2d[j, pl.ds(q, 16)]
            inc = (yv >= lo) & (yv < lo + CHUNK)
            dum = DUMMY + (yv & 63)
            idx[j, pl.ds(q, 16)] = jnp.where(inc, yv - lo, dum)
            return carry
        lax.fori_loop(0, (NSB * SB) // 16, icomp, 0)

        plsc.subcore_barrier()

        for j in range(NSB):
            with jax.named_scope("dma_gf"):
                cp = pltpu.async_copy(centers_hbm.at[y2d.at[j]], g_st, sem)
                pltpu.sync_copy(feat_hbm.at[pl.ds(s * SPT + j * SB, SB), :],
                                feat_st)
                cp.wait()

            def samp(i, acc):
                g16 = (i // 16) * 16
                qv = scale2d[j, pl.ds(g16, 16)]
                scv = lax.gather(
                    qv, jnp.full((16, 1), i - g16, jnp.int32),
                    dimension_numbers=lax.GatherDimensionNumbers(
                        offset_dims=(), collapsed_slice_dims=(0,),
                        start_index_map=(0,)),
                    slice_sizes=(1,),
                    mode=lax.GatherScatterMode.PROMISE_IN_BOUNDS)
                for qq in range(D // 16):
                    fv = feat_st[i, pl.ds(qq * 16, 16)]
                    gv = g_st[i, pl.ds(qq * 16, 16)]
                    dv = fv - gv
                    if phase == 0:
                        acc = acc + dv * dv
                    val_sb[i, pl.ds(qq * 16, 16)] = (gv - fv) * scv
                return acc
            with jax.named_scope("samp"):
                lacc = lax.fori_loop(0, SB, samp, lacc)

            with jax.named_scope("scat"):
                pltpu.sync_copy(val_sb, table.at[idx.at[j]], add=True)

        if phase == 0:
            lout[pl.ds(0, 16)] = lacc
            pltpu.sync_copy(lout, lsum.at[zidx], add=True)

        plsc.subcore_barrier()

        with jax.named_scope("copyout"):
            pass
        @pl.when(s < 15)
        def _():
            pltpu.sync_copy(table.at[pl.ds(s * 1568, 1568), :],
                            grad_hbm.at[pl.ds(lo + s * 1568, 1568), :])

        @pl.when(s == 15)
        def _():
            pltpu.sync_copy(table.at[pl.ds(23520, 1480), :],
                            grad_hbm.at[pl.ds(lo + 23520, 1480), :])

        if phase == 0:
            @pl.when((c == 0) & (s == 0))
            def _():
                pltpu.sync_copy(lsum, lread)
                lout[pl.ds(0, 16)] = lread[pl.ds(0, 16)] * LW
                pltpu.sync_copy(lout, loss_hbm)

        plsc.subcore_barrier()


_sc_call = pl.kernel(
    _body,
    out_type=(
        jax.ShapeDtypeStruct((C, D), jnp.float32),
        jax.ShapeDtypeStruct((16,), jnp.float32),
    ),
    mesh=plsc.VectorSubcoreMesh(core_axis_name="c", subcore_axis_name="s"),
    compiler_params=pltpu.CompilerParams(use_tc_tiling_on_sc=False),
    scratch_types=[
        pltpu.VMEM_SHARED((HIST_N,), jnp.int32),
        pltpu.VMEM_SHARED((TR, D), jnp.float32),
        pltpu.VMEM_SHARED((16,), jnp.float32),
        pltpu.VMEM((NSB, SB), jnp.int32),
        pltpu.VMEM((NSB, SB), jnp.int32),
        pltpu.VMEM((NSB, SB), jnp.float32),
        pltpu.VMEM((NSB, SB), jnp.int32),
        pltpu.VMEM((SB, D), jnp.float32),
        pltpu.VMEM((SB, D), jnp.float32),
        pltpu.VMEM((SB, D), jnp.float32),
        pltpu.VMEM((800,), jnp.int32),
        pltpu.VMEM((32, D), jnp.float32),
        pltpu.VMEM((SB,), jnp.int32),
        pltpu.VMEM((16,), jnp.int32),
        pltpu.VMEM((16,), jnp.float32),
        pltpu.VMEM((16,), jnp.float32),
        pltpu.VMEM((16,), jnp.float32),
        pltpu.SemaphoreType.DMA,
        pltpu.SemaphoreType.DMA,
    ],
)


def kernel(y, feat, centers):
    y2 = y.reshape(B // SB, SB)
    grad, lossv = _sc_call(y2, feat, centers)
    return lossv[0], grad


# speedup vs baseline: 1.1680x; 1.1680x over previous
import jax
import jax.numpy as jnp
from jax import lax
from jax.experimental import pallas as pl
from jax.experimental.pallas import tpu as pltpu
from jax.experimental.pallas import tpu_sc as plsc

B = 16384
D = 64
C = 100000
NS = 16
SPT = B // NS
SB = 32
NSB = SPT // SB
HIST_N = 102400
TR = 25088
DUMMY = 25000
CHUNK = 25000
LW = 0.005


def _copy_out(table, grad_hbm, s, lo):
    @pl.when(s < 15)
    def _():
        pltpu.sync_copy(table.at[pl.ds(s * 1568, 1568), :],
                        grad_hbm.at[pl.ds(lo + s * 1568, 1568), :])

    @pl.when(s == 15)
    def _():
        pltpu.sync_copy(table.at[pl.ds(23520, 1480), :],
                        grad_hbm.at[pl.ds(lo + 23520, 1480), :])


def _body(y_hbm, feat_hbm, centers_hbm, grad_hbm, loss_hbm, vstage,
          hist, table, lsum,
          y2d, hvm, scale2d,
          g_st0, g_st1, feat_st0, feat_st1, val0, val1,
          zb1, zb2, ones, zidx, zf32, lread, lout,
          gsem0, gsem1, fsem0, fsem1, vsem0, vsem1, zsem):
    c = lax.axis_index("c")
    s = lax.axis_index("s")
    lo_a = c * (2 * CHUNK)
    lo_b = lo_a + CHUNK
    g_st = (g_st0, g_st1)
    feat_st = (feat_st0, feat_st1)
    val = (val0, val1)
    gsem = (gsem0, gsem1)
    fsem = (fsem0, fsem1)
    vsem = (vsem0, vsem1)
    vbase = c * B + s * SPT

    def zf1(k, carry):
        zb1[pl.ds(k * 16, 16)] = jnp.zeros((16,), jnp.int32)
        return carry
    lax.fori_loop(0, 800 // 16, zf1, 0)

    def zf2(t, carry):
        r = t // 4
        q = (t % 4) * 16
        zb2[r, pl.ds(q, 16)] = jnp.zeros((16,), jnp.float32)
        return carry
    lax.fori_loop(0, 32 * 4, zf2, 0)

    def of(k, carry):
        ones[pl.ds(k * 16, 16)] = jnp.ones((16,), jnp.int32)
        return carry
    lax.fori_loop(0, SB // 16, of, 0)

    zidx[pl.ds(0, 16)] = jnp.zeros((16,), jnp.int32)
    zf32[pl.ds(0, 16)] = jnp.zeros((16,), jnp.float32)

    @pl.when(s == 0)
    def _():
        pltpu.sync_copy(zf32, lsum)

    with jax.named_scope("zinit"):
        cps = [pltpu.async_copy(zb1, hist.at[pl.ds(s * 6400 + k * 800, 800)],
                                zsem) for k in range(8)]
        cps += [pltpu.async_copy(zb2, table.at[pl.ds(s * 1568 + k * 32, 32), :],
                                 zsem) for k in range(49)]
        pltpu.sync_copy(y_hbm.at[pl.ds(s * NSB, NSB), :], y2d)
        for cp in cps:
            cp.wait()

    plsc.subcore_barrier()

    with jax.named_scope("histadd"):
        for j in range(NSB):
            pltpu.sync_copy(ones, hist.at[y2d.at[j]], add=True)

    plsc.subcore_barrier()

    with jax.named_scope("histgat"):
        for j in range(NSB):
            pltpu.sync_copy(hist.at[y2d.at[j]], hvm.at[j])

    def fcomp(t, carry):
        j = t // 2
        q = (t % 2) * 16
        hv = hvm[j, pl.ds(q, 16)]
        scale2d[j, pl.ds(q, 16)] = -1.0 / (1.0 + hv.astype(jnp.float32))
        return carry
    lax.fori_loop(0, (NSB * SB) // 16, fcomp, 0)

    def icompa(t, carry):
        j = t // 2
        q = (t % 2) * 16
        yv = y2d[j, pl.ds(q, 16)]
        inc = (yv >= lo_a) & (yv < lo_a + CHUNK)
        hvm[j, pl.ds(q, 16)] = jnp.where(inc, yv - lo_a, DUMMY + (yv & 63))
        return carry
    lax.fori_loop(0, (NSB * SB) // 16, icompa, 0)

    with jax.named_scope("dma_pre"):
        cpg = pltpu.async_copy(centers_hbm.at[y2d.at[0]], g_st[0], gsem[0])
        cpf = pltpu.async_copy(feat_hbm.at[pl.ds(s * SPT, SB), :],
                               feat_st[0], fsem[0])
    lacc = jnp.zeros((16,), jnp.float32)
    wcp = [None, None]
    for j in range(NSB):
        b = j & 1
        nb = 1 - b
        if j + 1 < NSB:
            with jax.named_scope("dma_next"):
                ncpg = pltpu.async_copy(centers_hbm.at[y2d.at[j + 1]],
                                        g_st[nb], gsem[nb])
                ncpf = pltpu.async_copy(
                    feat_hbm.at[pl.ds(s * SPT + (j + 1) * SB, SB), :],
                    feat_st[nb], fsem[nb])
        with jax.named_scope("dma_wait"):
            cpg.wait()
            cpf.wait()
            if j >= 2:
                wcp[b].wait()

        fs_ = feat_st[b]
        gs_ = g_st[b]
        vs_ = val[b]

        def samp(i, acc):
            g16 = (i // 16) * 16
            qv = scale2d[j, pl.ds(g16, 16)]
            scv = lax.gather(
                qv, jnp.full((16, 1), i - g16, jnp.int32),
                dimension_numbers=lax.GatherDimensionNumbers(
                    offset_dims=(), collapsed_slice_dims=(0,),
                    start_index_map=(0,)),
                slice_sizes=(1,),
                mode=lax.GatherScatterMode.PROMISE_IN_BOUNDS)
            for qq in range(D // 16):
                fv = fs_[i, pl.ds(qq * 16, 16)]
                gv = gs_[i, pl.ds(qq * 16, 16)]
                dv = fv - gv
                acc = acc + dv * dv
                vs_[i, pl.ds(qq * 16, 16)] = dv * scv
            return acc
        with jax.named_scope("samp"):
            lacc = lax.fori_loop(0, SB, samp, lacc)

        with jax.named_scope("scat"):
            pltpu.sync_copy(vs_, table.at[hvm.at[j]], add=True)
        with jax.named_scope("vstage_w"):
            wcp[b] = pltpu.async_copy(
                vs_, vstage.at[pl.ds(vbase + j * SB, SB), :], vsem[b])
        if j + 1 < NSB:
            cpg, cpf = ncpg, ncpf

    lout[pl.ds(0, 16)] = lacc
    pltpu.sync_copy(lout, lsum.at[zidx], add=True)

    def icompb(t, carry):
        j = t // 2
        q = (t % 2) * 16
        yv = y2d[j, pl.ds(q, 16)]
        inc = (yv >= lo_b) & (yv < lo_b + CHUNK)
        hvm[j, pl.ds(q, 16)] = jnp.where(inc, yv - lo_b, DUMMY + (yv & 63))
        return carry
    lax.fori_loop(0, (NSB * SB) // 16, icompb, 0)

    plsc.subcore_barrier()

    with jax.named_scope("copyout"):
        _copy_out(table, grad_hbm, s, lo_a)

    @pl.when((c == 0) & (s == 0))
    def _():
        pltpu.sync_copy(lsum, lread)
        lout[pl.ds(0, 16)] = lread[pl.ds(0, 16)] * LW
        pltpu.sync_copy(lout, loss_hbm)

    plsc.subcore_barrier()

    with jax.named_scope("ztab2"):
        cps = [pltpu.async_copy(zb2, table.at[pl.ds(s * 1568 + k * 32, 32), :],
                                zsem) for k in range(49)]
        wcp[0].wait()
        wcp[1].wait()
        for cp in cps:
            cp.wait()

    plsc.subcore_barrier()

    with jax.named_scope("phaseB"):
        cpr = pltpu.async_copy(vstage.at[pl.ds(vbase, SB), :], val[0],
                               fsem[0])
        for j in range(NSB):
            b = j & 1
            nb = 1 - b
            if j + 1 < NSB:
                ncpr = pltpu.async_copy(
                    vstage.at[pl.ds(vbase + (j + 1) * SB, SB), :],
                    val[nb], fsem[nb])
            cpr.wait()
            pltpu.sync_copy(val[b], table.at[hvm.at[j]], add=True)
            if j + 1 < NSB:
                cpr = ncpr

    plsc.subcore_barrier()

    with jax.named_scope("copyout2"):
        _copy_out(table, grad_hbm, s, lo_b)


_sc_call = pl.kernel(
    _body,
    out_type=(
        jax.ShapeDtypeStruct((C, D), jnp.float32),
        jax.ShapeDtypeStruct((16,), jnp.float32),
        jax.ShapeDtypeStruct((2 * B, D), jnp.float32),
    ),
    mesh=plsc.VectorSubcoreMesh(core_axis_name="c", subcore_axis_name="s"),
    compiler_params=pltpu.CompilerParams(use_tc_tiling_on_sc=False),
    scratch_types=[
        pltpu.VMEM_SHARED((HIST_N,), jnp.int32),
        pltpu.VMEM_SHARED((TR, D), jnp.float32),
        pltpu.VMEM_SHARED((16,), jnp.float32),
        pltpu.VMEM((NSB, SB), jnp.int32),
        pltpu.VMEM((NSB, SB), jnp.int32),
        pltpu.VMEM((NSB, SB), jnp.float32),
        pltpu.VMEM((SB, D), jnp.float32),
        pltpu.VMEM((SB, D), jnp.float32),
        pltpu.VMEM((SB, D), jnp.float32),
        pltpu.VMEM((SB, D), jnp.float32),
        pltpu.VMEM((SB, D), jnp.float32),
        pltpu.VMEM((SB, D), jnp.float32),
        pltpu.VMEM((800,), jnp.int32),
        pltpu.VMEM((32, D), jnp.float32),
        pltpu.VMEM((SB,), jnp.int32),
        pltpu.VMEM((16,), jnp.int32),
        pltpu.VMEM((16,), jnp.float32),
        pltpu.VMEM((16,), jnp.float32),
        pltpu.VMEM((16,), jnp.float32),
        pltpu.SemaphoreType.DMA,
        pltpu.SemaphoreType.DMA,
        pltpu.SemaphoreType.DMA,
        pltpu.SemaphoreType.DMA,
        pltpu.SemaphoreType.DMA,
        pltpu.SemaphoreType.DMA,
        pltpu.SemaphoreType.DMA,
    ],
)


def kernel(y, feat, centers):
    y2 = y.reshape(B // SB, SB)
    grad, lossv, _ = _sc_call(y2, feat, centers)
    return lossv[0], grad


# speedup vs baseline: 1.1930x; 1.0214x over previous
import jax
import jax.numpy as jnp
from jax import lax
from jax.experimental import pallas as pl
from jax.experimental.pallas import tpu as pltpu
from jax.experimental.pallas import tpu_sc as plsc

B = 16384
D = 64
C = 100000
NS = 16
SPT = B // NS
SB = 32
NSB = SPT // SB
HIST_N = 102400
TR = 25088
DUMMY = 25000
CHUNK = 25000
LW = 0.005


def _copy_out(table, grad_hbm, s, lo):
    @pl.when(s < 15)
    def _():
        pltpu.sync_copy(table.at[pl.ds(s * 1568, 1568), :],
                        grad_hbm.at[pl.ds(lo + s * 1568, 1568), :])

    @pl.when(s == 15)
    def _():
        pltpu.sync_copy(table.at[pl.ds(23520, 1480), :],
                        grad_hbm.at[pl.ds(lo + 23520, 1480), :])


def _body(y_hbm, feat_hbm, centers_hbm, grad_hbm, loss_hbm, vstage,
          hist, table, lsum,
          y2d, hvm, scale2d,
          g_st0, g_st1, feat_st0, feat_st1, val0, val1,
          zb1, zb2, ones, zidx, zf32, lread, lout,
          gsem0, gsem1, fsem0, fsem1, vsem0, vsem1, ssem0, ssem1, zsem):
    c = lax.axis_index("c")
    s = lax.axis_index("s")
    lo_a = c * (2 * CHUNK)
    lo_b = lo_a + CHUNK
    g_st = (g_st0, g_st1)
    feat_st = (feat_st0, feat_st1)
    val = (val0, val1)
    gsem = (gsem0, gsem1)
    fsem = (fsem0, fsem1)
    vsem = (vsem0, vsem1)
    vbase = c * B + s * SPT

    def zf1(k, carry):
        zb1[pl.ds(k * 16, 16)] = jnp.zeros((16,), jnp.int32)
        return carry
    lax.fori_loop(0, 800 // 16, zf1, 0)

    def zf2(t, carry):
        r = t // 4
        q = (t % 4) * 16
        zb2[r, pl.ds(q, 16)] = jnp.zeros((16,), jnp.float32)
        return carry
    lax.fori_loop(0, 32 * 4, zf2, 0)

    def of(k, carry):
        ones[pl.ds(k * 16, 16)] = jnp.ones((16,), jnp.int32)
        return carry
    lax.fori_loop(0, SB // 16, of, 0)

    zidx[pl.ds(0, 16)] = jnp.zeros((16,), jnp.int32)
    zf32[pl.ds(0, 16)] = jnp.zeros((16,), jnp.float32)

    @pl.when(s == 0)
    def _():
        pltpu.sync_copy(zf32, lsum)

    with jax.named_scope("zinit"):
        cps = [pltpu.async_copy(zb1, hist.at[pl.ds(s * 6400 + k * 800, 800)],
                                zsem) for k in range(8)]
        cps += [pltpu.async_copy(zb2, table.at[pl.ds(s * 1568 + k * 32, 32), :],
                                 zsem) for k in range(49)]
        pltpu.sync_copy(y_hbm.at[pl.ds(s * NSB, NSB), :], y2d)
        cpg = pltpu.async_copy(centers_hbm.at[y2d.at[0]], g_st[0], gsem[0])
        cpf = pltpu.async_copy(feat_hbm.at[pl.ds(s * SPT, SB), :],
                               feat_st[0], fsem[0])
        for cp in cps:
            cp.wait()

    plsc.subcore_barrier()

    with jax.named_scope("histadd"):
        for j in range(NSB):
            pltpu.sync_copy(ones, hist.at[y2d.at[j]], add=True)

    plsc.subcore_barrier()

    with jax.named_scope("histgat"):
        for j in range(NSB):
            pltpu.sync_copy(hist.at[y2d.at[j]], hvm.at[j])

    def fcomp(t, carry):
        j = t // 2
        q = (t % 2) * 16
        hv = hvm[j, pl.ds(q, 16)]
        scale2d[j, pl.ds(q, 16)] = -1.0 / (1.0 + hv.astype(jnp.float32))
        yv = y2d[j, pl.ds(q, 16)]
        inc = (yv >= lo_a) & (yv < lo_a + CHUNK)
        hvm[j, pl.ds(q, 16)] = jnp.where(inc, yv - lo_a, DUMMY + (yv & 63))
        return carry
    lax.fori_loop(0, (NSB * SB) // 16, fcomp, 0)

    lacc = jnp.zeros((16,), jnp.float32)
    wcp = [None, None]
    for j in range(NSB):
        b = j & 1
        nb = 1 - b
        if j + 1 < NSB:
            with jax.named_scope("dma_next"):
                ncpg = pltpu.async_copy(centers_hbm.at[y2d.at[j + 1]],
                                        g_st[nb], gsem[nb])
                ncpf = pltpu.async_copy(
                    feat_hbm.at[pl.ds(s * SPT + (j + 1) * SB, SB), :],
                    feat_st[nb], fsem[nb])
        with jax.named_scope("dma_wait"):
            cpg.wait()
            cpf.wait()
            if j >= 2:
                wcp[b].wait()

        fs_ = feat_st[b]
        gs_ = g_st[b]
        vs_ = val[b]

        def samp(i, acc):
            g16 = (i // 16) * 16
            qv = scale2d[j, pl.ds(g16, 16)]
            scv = lax.gather(
                qv, jnp.full((16, 1), i - g16, jnp.int32),
                dimension_numbers=lax.GatherDimensionNumbers(
                    offset_dims=(), collapsed_slice_dims=(0,),
                    start_index_map=(0,)),
                slice_sizes=(1,),
                mode=lax.GatherScatterMode.PROMISE_IN_BOUNDS)
            for qq in range(D // 16):
                fv = fs_[i, pl.ds(qq * 16, 16)]
                gv = gs_[i, pl.ds(qq * 16, 16)]
                dv = fv - gv
                acc = acc + dv * dv
                vs_[i, pl.ds(qq * 16, 16)] = dv * scv
            return acc
        with jax.named_scope("samp"):
            lacc = lax.fori_loop(0, SB, samp, lacc, unroll=2)

        with jax.named_scope("scat"):
            pltpu.sync_copy(vs_, table.at[hvm.at[j]], add=True)
        with jax.named_scope("vstage_w"):
            wcp[b] = pltpu.async_copy(
                vs_, vstage.at[pl.ds(vbase + j * SB, SB), :], vsem[b])
        if j + 1 < NSB:
            cpg, cpf = ncpg, ncpf

    lout[pl.ds(0, 16)] = lacc
    pltpu.sync_copy(lout, lsum.at[zidx], add=True)

    def icompb(t, carry):
        j = t // 2
        q = (t % 2) * 16
        yv = y2d[j, pl.ds(q, 16)]
        inc = (yv >= lo_b) & (yv < lo_b + CHUNK)
        hvm[j, pl.ds(q, 16)] = jnp.where(inc, yv - lo_b, DUMMY + (yv & 63))
        return carry
    lax.fori_loop(0, (NSB * SB) // 16, icompb, 0)

    plsc.subcore_barrier()

    with jax.named_scope("copyout"):
        _copy_out(table, grad_hbm, s, lo_a)

    @pl.when((c == 0) & (s == 0))
    def _():
        pltpu.sync_copy(lsum, lread)
        lout[pl.ds(0, 16)] = lread[pl.ds(0, 16)] * LW
        pltpu.sync_copy(lout, loss_hbm)

    with jax.named_scope("ztab2"):
        cps = [pltpu.async_copy(zb2, table.at[pl.ds(s * 1568 + k * 32, 32), :],
                                zsem) for k in range(49)]
        wcp[0].wait()
        wcp[1].wait()
        for cp in cps:
            cp.wait()

    plsc.subcore_barrier()

    with jax.named_scope("phaseB"):
        rbuf = (val[0], val[1], g_st[0], g_st[1])
        rsem = (fsem[0], fsem[1], gsem[0], gsem[1])
        ssem4 = (vsem[0], vsem[1], ssem0, ssem1)
        rcp = [None] * 4
        for j in range(4):
            rcp[j] = pltpu.async_copy(
                vstage.at[pl.ds(vbase + j * SB, SB), :], rbuf[j], rsem[j])
        for j in range(NSB):
            b = j & 3
            rcp[b].wait()
            pltpu.sync_copy(rbuf[b], table.at[hvm.at[j]], add=True)
            if j + 4 < NSB:
                rcp[b] = pltpu.async_copy(
                    vstage.at[pl.ds(vbase + (j + 4) * SB, SB), :],
                    rbuf[b], rsem[b])

    plsc.subcore_barrier()

    with jax.named_scope("copyout2"):
        _copy_out(table, grad_hbm, s, lo_b)


_sc_call = pl.kernel(
    _body,
    out_type=(
        jax.ShapeDtypeStruct((C, D), jnp.float32),
        jax.ShapeDtypeStruct((16,), jnp.float32),
        jax.ShapeDtypeStruct((2 * B, D), jnp.float32),
    ),
    mesh=plsc.VectorSubcoreMesh(core_axis_name="c", subcore_axis_name="s"),
    compiler_params=pltpu.CompilerParams(use_tc_tiling_on_sc=False),
    scratch_types=[
        pltpu.VMEM_SHARED((HIST_N,), jnp.int32),
        pltpu.VMEM_SHARED((TR, D), jnp.float32),
        pltpu.VMEM_SHARED((16,), jnp.float32),
        pltpu.VMEM((NSB, SB), jnp.int32),
        pltpu.VMEM((NSB, SB), jnp.int32),
        pltpu.VMEM((NSB, SB), jnp.float32),
        pltpu.VMEM((SB, D), jnp.float32),
        pltpu.VMEM((SB, D), jnp.float32),
        pltpu.VMEM((SB, D), jnp.float32),
        pltpu.VMEM((SB, D), jnp.float32),
        pltpu.VMEM((SB, D), jnp.float32),
        pltpu.VMEM((SB, D), jnp.float32),
        pltpu.VMEM((800,), jnp.int32),
        pltpu.VMEM((32, D), jnp.float32),
        pltpu.VMEM((SB,), jnp.int32),
        pltpu.VMEM((16,), jnp.int32),
        pltpu.VMEM((16,), jnp.float32),
        pltpu.VMEM((16,), jnp.float32),
        pltpu.VMEM((16,), jnp.float32),
        pltpu.SemaphoreType.DMA,
        pltpu.SemaphoreType.DMA,
        pltpu.SemaphoreType.DMA,
        pltpu.SemaphoreType.DMA,
        pltpu.SemaphoreType.DMA,
        pltpu.SemaphoreType.DMA,
        pltpu.SemaphoreType.DMA,
        pltpu.SemaphoreType.DMA,
        pltpu.SemaphoreType.DMA,
    ],
)


def kernel(y, feat, centers):
    y2 = y.reshape(B // SB, SB)
    grad, lossv, _ = _sc_call(y2, feat, centers)
    return lossv[0], grad
